# baseline (speedup 1.0000x reference)
import functools

import jax
import jax.numpy as jnp
from jax import lax
from jax.experimental import pallas as pl
from jax.experimental.pallas import tpu as pltpu
from jax.experimental.pallas import tpu_sc as plsc

B, C, H, W = 16, 64, 128, 128
HW = H * W
N = 8192
NC, NS, L = 2, 16, 16
NW = NC * NS
NPW = N // NW
EPW = NPW * C
CHUNK = 128
NCHUNK = EPW // CHUNK
NGRP = 8


def _make_sc_call():
    mesh = plsc.VectorSubcoreMesh(core_axis_name="c", subcore_axis_name="s")

    @functools.partial(
        pl.kernel,
        mesh=mesh,
        out_type=jax.ShapeDtypeStruct((NW, C), jnp.float32),
        scratch_types=[
            pltpu.VMEM((NPW,), jnp.int32),
            pltpu.VMEM((NPW,), jnp.int32),
            pltpu.VMEM((NPW,), jnp.int32),
            pltpu.VMEM((EPW,), jnp.int32),
            pltpu.VMEM((EPW,), jnp.float32),
            pltpu.VMEM((NPW, C), jnp.float32),
            pltpu.VMEM((C,), jnp.float32),
            pltpu.SemaphoreType.DMA,
        ] + [pltpu.SemaphoreType.DMA] * NGRP,
        compiler_params=pltpu.CompilerParams(skip_device_barrier=True),
    )
    def sc_loss(out_flat, tgt_2d, ind_hbm, batch_hbm, out_hbm,
                ind_v, batch_v, base_v, idx_v, val_v, tgt_v, part_v,
                sem_t, *sems):
        wid = lax.axis_index("s") * NC + lax.axis_index("c")
        obj0 = wid * NPW
        ngrp = NGRP
        cpg = NCHUNK // ngrp

        tgt_cp = pltpu.make_async_copy(
            tgt_2d.at[pl.ds(obj0, NPW)], tgt_v, sem_t)
        tgt_cp.start()
        pltpu.sync_copy(ind_hbm.at[pl.ds(obj0, NPW)], ind_v)
        pltpu.sync_copy(batch_hbm.at[pl.ds(obj0, NPW)], batch_v)

        def mk_base(g, _):
            bv = batch_v[pl.ds(g * L, L)]
            iv = ind_v[pl.ds(g * L, L)]
            base_v[pl.ds(g * L, L)] = bv * (C * HW) + iv
            return 0
        lax.fori_loop(0, NPW // L, mk_base, 0)

        ramps = [lax.iota(jnp.int32, L) * HW + k * L * HW for k in range(C // L)]
        gdn = lax.GatherDimensionNumbers(
            offset_dims=(), collapsed_slice_dims=(0,), start_index_map=(0,))
        opc = CHUNK // C

        def gather_cp(g, sem):
            return pltpu.make_async_copy(
                out_flat.at[idx_v.at[pl.ds(g * CHUNK, CHUNK)]],
                val_v.at[pl.ds(g * CHUNK, CHUNK)],
                sem,
            )

        for j in range(ngrp):
            def fire(g, _, _j=j):
                n0 = g * opc
                gv = n0 // L
                bv = base_v[pl.ds(gv * L, L)]
                for o in range(opc):
                    n = n0 + o
                    lane = lax.rem(n, jnp.int32(L))
                    bb = lax.gather(
                        bv, lane * jnp.ones((L, 1), jnp.int32),
                        gdn, slice_sizes=(1,),
                        mode=lax.GatherScatterMode.PROMISE_IN_BOUNDS)
                    for k in range(C // L):
                        idx_v[pl.ds(n * C + k * L, L)] = bb + ramps[k]
                gather_cp(g, sems[_j]).start()
                return 0
            lax.fori_loop(j * cpg, (j + 1) * cpg, fire, 0)

        tgt_cp.wait()

        inv = jnp.float32(1.0 / (N + 0.0001))
        zeros = tuple(jnp.zeros((L,), jnp.float32) for _ in range(C // L))
        accs = zeros
        for j in range(ngrp):
            def drain(g, _, _j=j):
                gather_cp(g, sems[_j]).wait()
                return 0
            lax.fori_loop(j * cpg, (j + 1) * cpg, drain, 0)

            def acc_fn(n, a):
                outs = []
                for k in range(C // L):
                    v = val_v[pl.ds(n * C + k * L, L)]
                    t = tgt_v[n, pl.ds(k * L, L)]
                    outs.append(a[k] + jnp.abs(v - t))
                return tuple(outs)
            accs = lax.fori_loop(j * cpg * opc, (j + 1) * cpg * opc,
                                 acc_fn, accs)

        for k in range(C // L):
            part_v[pl.ds(k * L, L)] = accs[k] * inv
        pltpu.sync_copy(part_v, out_hbm.at[wid])

    return sc_loss


_sc_loss = _make_sc_call()


@jax.jit
def kernel(output, target, ind, batch):
    out_flat = output.reshape(-1)
    ind32 = ind.astype(jnp.int32)
    batch32 = batch.astype(jnp.int32)
    partials = _sc_loss(out_flat, target, ind32, batch32)
    return partials.sum(axis=0)


# speedup vs baseline: 1.0067x; 1.0067x over previous
import functools

import jax
import jax.numpy as jnp
from jax import lax
from jax.experimental import pallas as pl
from jax.experimental.pallas import tpu as pltpu
from jax.experimental.pallas import tpu_sc as plsc

B, C, H, W = 16, 64, 128, 128
HW = H * W
N = 8192
NC, NS, L = 2, 16, 16
NW = NC * NS
NPW = N // NW
EPW = NPW * C
CHUNK = 128
NCHUNK = EPW // CHUNK
NGRP = 8


def _make_sc_call():
    mesh = plsc.VectorSubcoreMesh(core_axis_name="c", subcore_axis_name="s")

    @functools.partial(
        pl.kernel,
        mesh=mesh,
        out_type=jax.ShapeDtypeStruct((NW, C), jnp.float32),
        scratch_types=[
            pltpu.VMEM((NPW,), jnp.int32),
            pltpu.VMEM((NPW,), jnp.int32),
            pltpu.VMEM((NPW,), jnp.int32),
            pltpu.VMEM((EPW,), jnp.int32),
            pltpu.VMEM((EPW,), jnp.float32),
            pltpu.VMEM((NPW, C), jnp.float32),
            pltpu.VMEM((C,), jnp.float32),
            pltpu.SemaphoreType.DMA,
        ] + [pltpu.SemaphoreType.DMA] * NGRP,
    )
    def sc_loss(out_flat, tgt_2d, ind_hbm, batch_hbm, out_hbm,
                ind_v, batch_v, base_v, idx_v, val_v, tgt_v, part_v,
                sem_t, *sems):
        wid = lax.axis_index("s") * NC + lax.axis_index("c")
        obj0 = wid * NPW
        ngrp = NGRP
        cpg = NCHUNK // ngrp

        tgt_cp = pltpu.make_async_copy(
            tgt_2d.at[pl.ds(obj0, NPW)], tgt_v, sem_t)
        tgt_cp.start()
        ind_cp = pltpu.make_async_copy(
            ind_hbm.at[pl.ds(obj0, NPW)], ind_v, sems[0])
        bat_cp = pltpu.make_async_copy(
            batch_hbm.at[pl.ds(obj0, NPW)], batch_v, sems[1])
        ind_cp.start()
        bat_cp.start()
        ind_cp.wait()
        bat_cp.wait()

        def mk_base(g, _):
            bv = batch_v[pl.ds(g * L, L)]
            iv = ind_v[pl.ds(g * L, L)]
            base_v[pl.ds(g * L, L)] = bv * (C * HW) + iv
            return 0
        lax.fori_loop(0, NPW // L, mk_base, 0)

        ramps = [lax.iota(jnp.int32, L) * HW + k * L * HW for k in range(C // L)]
        gdn = lax.GatherDimensionNumbers(
            offset_dims=(), collapsed_slice_dims=(0,), start_index_map=(0,))
        opc = CHUNK // C

        def gather_cp(g, sem):
            return pltpu.make_async_copy(
                out_flat.at[idx_v.at[pl.ds(g * CHUNK, CHUNK)]],
                val_v.at[pl.ds(g * CHUNK, CHUNK)],
                sem,
            )

        for j in range(ngrp):
            def fire(g, _, _j=j):
                n0 = g * opc
                gv = n0 // L
                bv = base_v[pl.ds(gv * L, L)]
                for o in range(opc):
                    n = n0 + o
                    lane = lax.rem(n, jnp.int32(L))
                    bb = lax.gather(
                        bv, lane * jnp.ones((L, 1), jnp.int32),
                        gdn, slice_sizes=(1,),
                        mode=lax.GatherScatterMode.PROMISE_IN_BOUNDS)
                    for k in range(C // L):
                        idx_v[pl.ds(n * C + k * L, L)] = bb + ramps[k]
                gather_cp(g, sems[_j]).start()
                return 0
            lax.fori_loop(j * cpg, (j + 1) * cpg, fire, 0)

        tgt_cp.wait()

        inv = jnp.float32(1.0 / (N + 0.0001))
        zeros = tuple(jnp.zeros((L,), jnp.float32) for _ in range(C // L))
        accs = zeros
        for j in range(ngrp):
            def drain(g, _, _j=j):
                gather_cp(g, sems[_j]).wait()
                return 0
            lax.fori_loop(j * cpg, (j + 1) * cpg, drain, 0)

            def acc_fn(n, a):
                outs = []
                for k in range(C // L):
                    v = val_v[pl.ds(n * C + k * L, L)]
                    t = tgt_v[n, pl.ds(k * L, L)]
                    outs.append(a[k] + jnp.abs(v - t))
                return tuple(outs)
            accs = lax.fori_loop(j * cpg * opc, (j + 1) * cpg * opc,
                                 acc_fn, accs)

        for k in range(C // L):
            part_v[pl.ds(k * L, L)] = accs[k] * inv
        pltpu.sync_copy(part_v, out_hbm.at[wid])

    return sc_loss


_sc_loss = _make_sc_call()


@jax.jit
def kernel(output, target, ind, batch):
    out_flat = output.reshape(-1)
    ind32 = ind.astype(jnp.int32)
    batch32 = batch.astype(jnp.int32)
    partials = _sc_loss(out_flat, target, ind32, batch32)
    return partials.sum(axis=0)
